# baseline (speedup 1.0000x reference)
import functools

import jax
import jax.numpy as jnp
from jax import lax
from jax.experimental import pallas as pl
from jax.experimental.pallas import tpu as pltpu
from jax.experimental.pallas import tpu_sc as plsc

_K = 1024
_D = 64
_T = 32 * 576
_TH = _T // 2
_TB = 2304
_NBLK = _TH // _TB

_NC = 2
_NS = 16
_NW = _NC * _NS
_BPW = _TH // _NW
_ICH = 96
_NCH = _BPW // _ICH


def _make_vq_body(final: bool):
    def _vq_body(x_ref, et_ref, hist_in_ref, sse_in_ref,
                 idx_ref, hist_out_ref, sse_out_ref, loss_ref, perp_ref,
                 hist_ref, sse_ref):
        i = pl.program_id(0)
        x = x_ref[...]
        et = et_ref[...]
        x2 = jnp.sum(x * x, axis=1, keepdims=True)
        s2 = jnp.dot(x, et + et, preferred_element_type=jnp.float32)
        e2 = jnp.sum(et * et, axis=0, keepdims=True)
        d = x2 - s2 + e2
        m = jnp.min(d, axis=1, keepdims=True)
        eqf = (d == m).astype(jnp.float32)
        hist_blk = jnp.sum(eqf, axis=0, keepdims=True)
        s_tot = jnp.sum(hist_blk)

        @pl.when(i == 0)
        def _init():
            hist_ref[...] = hist_in_ref[...]
            sse_ref[...] = sse_in_ref[...]

        iota_row = lax.broadcasted_iota(jnp.int32, (1, _K), 1).astype(jnp.float32)

        @pl.when(s_tot == float(_TB))
        def _fast():
            idxf = jnp.sum(eqf * iota_row, axis=1, keepdims=True)
            idx_ref[...] = idxf.astype(jnp.int32)
            hist_ref[...] += hist_blk

        @pl.when(s_tot != float(_TB))
        def _ties():
            iota = lax.broadcasted_iota(jnp.int32, d.shape, 1)
            idx = jnp.min(jnp.where(d == m, iota, _K), axis=1, keepdims=True)
            idx_ref[...] = idx
            oh = (iota == idx).astype(jnp.float32)
            hist_ref[...] += jnp.sum(oh, axis=0, keepdims=True)

        sse_ref[...] += jnp.sum(m, keepdims=True)

        @pl.when(i == _NBLK - 1)
        def _flush():
            hist_out_ref[...] = hist_ref[...]
            sse_out_ref[...] = sse_ref[...]
            if final:
                loss_ref[...] = sse_ref[...] * (1.25 / (_T * _D))
                p = hist_ref[...] * (1.0 / _T)
                ent = jnp.sum(p * jnp.log(p + 1e-10), axis=1, keepdims=True)
                perp_ref[...] = jnp.exp(-ent)

    return _vq_body


def _vq_call(flat_half, et, hist_in, sse_in, final):
    return pl.pallas_call(
        _make_vq_body(final),
        grid=(_NBLK,),
        in_specs=[
            pl.BlockSpec((_TB, _D), lambda i: (i, 0)),
            pl.BlockSpec((_D, _K), lambda i: (0, 0)),
            pl.BlockSpec((1, _K), lambda i: (0, 0)),
            pl.BlockSpec((1, 1), lambda i: (0, 0)),
        ],
        out_specs=[
            pl.BlockSpec((_TB, 1), lambda i: (i, 0)),
            pl.BlockSpec((1, _K), lambda i: (0, 0)),
            pl.BlockSpec((1, 1), lambda i: (0, 0)),
            pl.BlockSpec((1, 1), lambda i: (0, 0)),
            pl.BlockSpec((1, 1), lambda i: (0, 0)),
        ],
        out_shape=[
            jax.ShapeDtypeStruct((_TH, 1), jnp.int32),
            jax.ShapeDtypeStruct((1, _K), jnp.float32),
            jax.ShapeDtypeStruct((1, 1), jnp.float32),
            jax.ShapeDtypeStruct((1, 1), jnp.float32),
            jax.ShapeDtypeStruct((1, 1), jnp.float32),
        ],
        scratch_shapes=[
            pltpu.VMEM((1, _K), jnp.float32),
            pltpu.VMEM((1, 1), jnp.float32),
        ],
    )(flat_half, et, hist_in, sse_in)


@functools.lru_cache(maxsize=1)
def _make_sc_gather():
    mesh = plsc.VectorSubcoreMesh(core_axis_name="c", subcore_axis_name="s")

    @functools.partial(
        pl.kernel,
        mesh=mesh,
        out_type=jax.ShapeDtypeStruct((_TH, _D), jnp.float32),
        scratch_types=[
            pltpu.VMEM((_NCH, _ICH), jnp.int32),
            pltpu.VMEM((_BPW, _D), jnp.float32),
            pltpu.SemaphoreType.DMA,
            pltpu.SemaphoreType.DMA,
        ],
        compiler_params=pltpu.CompilerParams(use_tc_tiling_on_sc=False),
    )
    def gather_k(table_hbm, idx_hbm, out_hbm, idx_v, rows_v, gsem, osem):
        wid = lax.axis_index("s") * _NC + lax.axis_index("c")
        base = wid * _BPW
        pltpu.sync_copy(idx_hbm.at[wid], idx_v)
        gathers = [
            pltpu.async_copy(
                table_hbm.at[idx_v.at[j]],
                rows_v.at[pl.ds(j * _ICH, _ICH)],
                gsem,
            )
            for j in range(_NCH)
        ]
        outs = []
        for j, g in enumerate(gathers):
            g.wait()
            outs.append(pltpu.async_copy(
                rows_v.at[pl.ds(j * _ICH, _ICH)],
                out_hbm.at[pl.ds(base + j * _ICH, _ICH)],
                osem,
            ))
        for o in outs:
            o.wait()

    return gather_k


def kernel(inputs, embedding):
    flat = inputs.reshape(_T, _D)
    et = embedding.T
    hist0 = jnp.zeros((1, _K), jnp.float32)
    sse0 = jnp.zeros((1, 1), jnp.float32)
    idx1, hist1, sse1, _, _ = _vq_call(flat[:_TH], et, hist0, sse0, False)
    idx2, hist2, sse2, loss, perp = _vq_call(flat[_TH:], et, hist1, sse1, True)
    gather = _make_sc_gather()
    q1 = gather(embedding, idx1.reshape(_NW, _NCH, _ICH))
    q2 = gather(embedding, idx2.reshape(_NW, _NCH, _ICH))
    q = jnp.concatenate([q1, q2], axis=0)
    return (
        q.reshape(inputs.shape),
        loss.reshape(()),
        perp.reshape(()),
    )


# speedup vs baseline: 1.2561x; 1.2561x over previous
import functools

import jax
import jax.numpy as jnp
from jax import lax
from jax.experimental import pallas as pl
from jax.experimental.pallas import tpu as pltpu
from jax.experimental.pallas import tpu_sc as plsc

_K = 1024
_D = 64
_T = 32 * 576
_TB = 2304
_NBLK = _T // _TB

_NC = 2
_NS = 16
_NW = _NC * _NS
_BPW = _T // _NW
_ICH = 96
_NCH = _BPW // _ICH


def _vq_body(x_ref, et_ref, idx_ref, loss_ref, perp_ref, hist_ref, sse_ref):
    i = pl.program_id(0)
    x = x_ref[...]
    et = et_ref[...]
    x2 = jnp.sum(x * x, axis=1, keepdims=True)
    s2 = jnp.dot(x, et + et, preferred_element_type=jnp.float32)
    e2 = jnp.sum(et * et, axis=0, keepdims=True)
    d = x2 - s2 + e2
    m = jnp.min(d, axis=1, keepdims=True)
    eqf = (d == m).astype(jnp.float32)
    hist_blk = jnp.sum(eqf, axis=0, keepdims=True)
    s_tot = jnp.sum(hist_blk)

    @pl.when(i == 0)
    def _init():
        hist_ref[...] = jnp.zeros_like(hist_ref)
        sse_ref[...] = jnp.zeros_like(sse_ref)

    iota_row = lax.broadcasted_iota(jnp.int32, (1, _K), 1).astype(jnp.float32)

    @pl.when(s_tot == float(_TB))
    def _fast():
        idxf = jnp.sum(eqf * iota_row, axis=1, keepdims=True)
        idx_ref[...] = idxf.astype(jnp.int32).reshape(1, _TB // 128, 128)
        hist_ref[...] += hist_blk

    @pl.when(s_tot != float(_TB))
    def _ties():
        iota = lax.broadcasted_iota(jnp.int32, d.shape, 1)
        idx = jnp.min(jnp.where(d == m, iota, _K), axis=1, keepdims=True)
        idx_ref[...] = idx.reshape(1, _TB // 128, 128)
        oh = (iota == idx).astype(jnp.float32)
        hist_ref[...] += jnp.sum(oh, axis=0, keepdims=True)

    sse_ref[...] += jnp.sum(m, keepdims=True)

    @pl.when(i == _NBLK - 1)
    def _fini():
        loss_ref[...] = sse_ref[...] * (1.25 / (_T * _D))
        p = hist_ref[...] * (1.0 / _T)
        ent = jnp.sum(p * jnp.log(p + 1e-10), axis=1, keepdims=True)
        perp_ref[...] = jnp.exp(-ent)


def _vq_call(flat, et):
    return pl.pallas_call(
        _vq_body,
        grid=(_NBLK,),
        in_specs=[
            pl.BlockSpec((_TB, _D), lambda i: (i, 0)),
            pl.BlockSpec((_D, _K), lambda i: (0, 0)),
        ],
        out_specs=[
            pl.BlockSpec((1, _TB // 128, 128), lambda i: (i, 0, 0)),
            pl.BlockSpec((1, 1), lambda i: (0, 0)),
            pl.BlockSpec((1, 1), lambda i: (0, 0)),
        ],
        out_shape=[
            jax.ShapeDtypeStruct((_NBLK, _TB // 128, 128), jnp.int32),
            jax.ShapeDtypeStruct((1, 1), jnp.float32),
            jax.ShapeDtypeStruct((1, 1), jnp.float32),
        ],
        scratch_shapes=[
            pltpu.VMEM((1, _K), jnp.float32),
            pltpu.VMEM((1, 1), jnp.float32),
        ],
    )(flat, et)


@functools.lru_cache(maxsize=1)
def _make_sc_gather():
    mesh = plsc.VectorSubcoreMesh(core_axis_name="c", subcore_axis_name="s")

    @functools.partial(
        pl.kernel,
        mesh=mesh,
        out_type=jax.ShapeDtypeStruct((_T, _D), jnp.float32),
        scratch_types=[
            pltpu.VMEM((_NCH, _ICH), jnp.int32),
            pltpu.VMEM((_BPW, _D), jnp.float32),
            pltpu.SemaphoreType.DMA,
        ],
        compiler_params=pltpu.CompilerParams(use_tc_tiling_on_sc=False),
    )
    def gather_k(table_hbm, idx_hbm, out_hbm, idx_v, rows_v, sem):
        wid = lax.axis_index("s") * _NC + lax.axis_index("c")
        base = wid * _BPW
        pltpu.sync_copy(idx_hbm.at[wid], idx_v)
        copies = [
            pltpu.async_copy(
                table_hbm.at[idx_v.at[j]],
                rows_v.at[pl.ds(j * _ICH, _ICH)],
                sem,
            )
            for j in range(_NCH)
        ]
        for c in copies:
            c.wait()
        pltpu.sync_copy(rows_v, out_hbm.at[pl.ds(base, _BPW)])

    return gather_k


def kernel(inputs, embedding):
    flat = inputs.reshape(_T, _D)
    et = embedding.T
    idx, loss, perp = _vq_call(flat, et)
    idx_w = idx.reshape(_NW, _NCH, _ICH)
    quantized = _make_sc_gather()(embedding, idx_w)
    return (
        quantized.reshape(inputs.shape),
        loss.reshape(()),
        perp.reshape(()),
    )


# speedup vs baseline: 1.2662x; 1.0080x over previous
import functools

import jax
import jax.numpy as jnp
from jax import lax
from jax.experimental import pallas as pl
from jax.experimental.pallas import tpu as pltpu
from jax.experimental.pallas import tpu_sc as plsc

_K = 1024
_D = 64
_T = 32 * 576
_TB = 2304
_NBLK = _T // _TB

_NC = 2
_NS = 16
_NW = _NC * _NS
_BPW = _T // _NW
_ICH = 96
_NCH = _BPW // _ICH


def _vq_body(x_ref, et_ref, idx_ref, loss_ref, perp_ref, hist_ref, sse_ref):
    i = pl.program_id(0)
    x = x_ref[...]
    et = et_ref[...]
    x2 = jnp.sum(x * x, axis=1, keepdims=True)
    s2 = jnp.dot(x, et + et, preferred_element_type=jnp.float32)
    e2 = jnp.sum(et * et, axis=0, keepdims=True)
    d = x2 - s2 + e2
    m = jnp.min(d, axis=1, keepdims=True)
    eqf = (d == m).astype(jnp.float32)
    hist_blk = jnp.sum(eqf, axis=0, keepdims=True)
    s_tot = jnp.sum(hist_blk)

    @pl.when(i == 0)
    def _init():
        hist_ref[...] = jnp.zeros_like(hist_ref)
        sse_ref[...] = jnp.zeros_like(sse_ref)

    iota_row = lax.broadcasted_iota(jnp.int32, (1, _K), 1).astype(jnp.float32)

    @pl.when(s_tot == float(_TB))
    def _fast():
        idxf = jnp.sum(eqf * iota_row, axis=1, keepdims=True)
        idx_ref[...] = idxf.astype(jnp.int32).reshape(1, _TB // 128, 128)
        hist_ref[...] += hist_blk

    @pl.when(s_tot != float(_TB))
    def _ties():
        iota = lax.broadcasted_iota(jnp.int32, d.shape, 1)
        idx = jnp.min(jnp.where(d == m, iota, _K), axis=1, keepdims=True)
        idx_ref[...] = idx.reshape(1, _TB // 128, 128)
        oh = (iota == idx).astype(jnp.float32)
        hist_ref[...] += jnp.sum(oh, axis=0, keepdims=True)

    sse_ref[...] += jnp.sum(m, keepdims=True)

    @pl.when(i == _NBLK - 1)
    def _fini():
        loss_ref[...] = sse_ref[...] * (1.25 / (_T * _D))
        p = hist_ref[...] * (1.0 / _T)
        ent = jnp.sum(p * jnp.log(p + 1e-10), axis=1, keepdims=True)
        perp_ref[...] = jnp.exp(-ent)


def _vq_call(flat, et):
    return pl.pallas_call(
        _vq_body,
        grid=(_NBLK,),
        in_specs=[
            pl.BlockSpec((_TB, _D), lambda i: (i, 0)),
            pl.BlockSpec((_D, _K), lambda i: (0, 0)),
        ],
        out_specs=[
            pl.BlockSpec((1, _TB // 128, 128), lambda i: (i, 0, 0)),
            pl.BlockSpec((1, 1), lambda i: (0, 0)),
            pl.BlockSpec((1, 1), lambda i: (0, 0)),
        ],
        out_shape=[
            jax.ShapeDtypeStruct((_NBLK, _TB // 128, 128), jnp.int32),
            jax.ShapeDtypeStruct((1, 1), jnp.float32),
            jax.ShapeDtypeStruct((1, 1), jnp.float32),
        ],
        scratch_shapes=[
            pltpu.VMEM((1, _K), jnp.float32),
            pltpu.VMEM((1, 1), jnp.float32),
        ],
    )(flat, et)


@functools.lru_cache(maxsize=1)
def _make_sc_gather():
    mesh = plsc.VectorSubcoreMesh(core_axis_name="c", subcore_axis_name="s")

    @functools.partial(
        pl.kernel,
        mesh=mesh,
        out_type=jax.ShapeDtypeStruct((_NW, _BPW, _D), jnp.float32),
        scratch_types=[
            pltpu.VMEM((_NCH, _ICH), jnp.int32),
            pltpu.VMEM((_BPW, _D), jnp.float32),
            pltpu.SemaphoreType.DMA,
        ],
        compiler_params=pltpu.CompilerParams(use_tc_tiling_on_sc=False),
    )
    def gather_k(table_hbm, idx_hbm, out_hbm, idx_v, rows_v, sem):
        wid = lax.axis_index("s") * _NC + lax.axis_index("c")
        pltpu.sync_copy(idx_hbm.at[wid], idx_v)
        copies = [
            pltpu.async_copy(
                table_hbm.at[idx_v.at[j]],
                rows_v.at[pl.ds(j * _ICH, _ICH)],
                sem,
            )
            for j in range(_NCH)
        ]
        for c in copies:
            c.wait()
        pltpu.sync_copy(rows_v, out_hbm.at[wid])

    return gather_k


def kernel(inputs, embedding):
    flat = inputs.reshape(_T, _D)
    et = embedding.T
    idx, loss, perp = _vq_call(flat, et)
    idx_w = idx.reshape(_NW, _NCH, _ICH)
    quantized = _make_sc_gather()(embedding, idx_w)
    return (
        quantized,
        loss.reshape(()),
        perp.reshape(()),
    )


# speedup vs baseline: 1.2965x; 1.0239x over previous
import functools

import jax
import jax.numpy as jnp
from jax import lax
from jax.experimental import pallas as pl
from jax.experimental.pallas import tpu as pltpu
from jax.experimental.pallas import tpu_sc as plsc

_K = 1024
_D = 64
_T = 32 * 576
_TB = 4608
_NBLK = _T // _TB

_NC = 2
_NS = 16
_NW = _NC * _NS
_BPW = _T // _NW
_ICH = 96
_NCH = _BPW // _ICH


def _vq_body(x_ref, et_ref, idx_ref, loss_ref, perp_ref, hist_ref, sse_ref):
    i = pl.program_id(0)
    x = x_ref[...]
    et = et_ref[...]
    x2 = jnp.sum(x * x, axis=1, keepdims=True)
    s2 = jnp.dot(x, et + et, preferred_element_type=jnp.float32)
    e2 = jnp.sum(et * et, axis=0, keepdims=True)
    d = x2 - s2 + e2
    m = jnp.min(d, axis=1, keepdims=True)
    eqf = (d == m).astype(jnp.float32)
    hist_blk = jnp.sum(eqf, axis=0, keepdims=True)
    s_tot = jnp.sum(hist_blk)

    @pl.when(i == 0)
    def _init():
        hist_ref[...] = jnp.zeros_like(hist_ref)
        sse_ref[...] = jnp.zeros_like(sse_ref)

    iota_row = lax.broadcasted_iota(jnp.int32, (1, _K), 1).astype(jnp.float32)

    @pl.when(s_tot == float(_TB))
    def _fast():
        idxf = jnp.sum(eqf * iota_row, axis=1, keepdims=True)
        idx_ref[...] = idxf.astype(jnp.int32).reshape(1, _TB // 128, 128)
        hist_ref[...] += hist_blk

    @pl.when(s_tot != float(_TB))
    def _ties():
        iota = lax.broadcasted_iota(jnp.int32, d.shape, 1)
        idx = jnp.min(jnp.where(d == m, iota, _K), axis=1, keepdims=True)
        idx_ref[...] = idx.reshape(1, _TB // 128, 128)
        oh = (iota == idx).astype(jnp.float32)
        hist_ref[...] += jnp.sum(oh, axis=0, keepdims=True)

    sse_ref[...] += jnp.sum(m, keepdims=True)

    @pl.when(i == _NBLK - 1)
    def _fini():
        loss_ref[...] = sse_ref[...] * (1.25 / (_T * _D))
        p = hist_ref[...] * (1.0 / _T)
        ent = jnp.sum(p * jnp.log(p + 1e-10), axis=1, keepdims=True)
        perp_ref[...] = jnp.exp(-ent)


def _vq_call(flat, et):
    return pl.pallas_call(
        _vq_body,
        grid=(_NBLK,),
        in_specs=[
            pl.BlockSpec((_TB, _D), lambda i: (i, 0)),
            pl.BlockSpec((_D, _K), lambda i: (0, 0)),
        ],
        out_specs=[
            pl.BlockSpec((1, _TB // 128, 128), lambda i: (i, 0, 0)),
            pl.BlockSpec((1, 1), lambda i: (0, 0)),
            pl.BlockSpec((1, 1), lambda i: (0, 0)),
        ],
        out_shape=[
            jax.ShapeDtypeStruct((_NBLK, _TB // 128, 128), jnp.int32),
            jax.ShapeDtypeStruct((1, 1), jnp.float32),
            jax.ShapeDtypeStruct((1, 1), jnp.float32),
        ],
        scratch_shapes=[
            pltpu.VMEM((1, _K), jnp.float32),
            pltpu.VMEM((1, 1), jnp.float32),
        ],
    )(flat, et)


@functools.lru_cache(maxsize=1)
def _make_sc_gather():
    mesh = plsc.VectorSubcoreMesh(core_axis_name="c", subcore_axis_name="s")

    @functools.partial(
        pl.kernel,
        mesh=mesh,
        out_type=jax.ShapeDtypeStruct((_NW, _BPW, _D), jnp.float32),
        scratch_types=[
            pltpu.VMEM((_NCH, _ICH), jnp.int32),
            pltpu.VMEM((_BPW, _D), jnp.float32),
            pltpu.SemaphoreType.DMA,
        ],
        compiler_params=pltpu.CompilerParams(use_tc_tiling_on_sc=False),
    )
    def gather_k(table_hbm, idx_hbm, out_hbm, idx_v, rows_v, sem):
        wid = lax.axis_index("s") * _NC + lax.axis_index("c")
        pltpu.sync_copy(idx_hbm.at[wid], idx_v)
        copies = [
            pltpu.async_copy(
                table_hbm.at[idx_v.at[j]],
                rows_v.at[pl.ds(j * _ICH, _ICH)],
                sem,
            )
            for j in range(_NCH)
        ]
        for c in copies:
            c.wait()
        pltpu.sync_copy(rows_v, out_hbm.at[wid])

    return gather_k


def kernel(inputs, embedding):
    flat = inputs.reshape(_T, _D)
    et = embedding.T
    idx, loss, perp = _vq_call(flat, et)
    idx_w = idx.reshape(_NW, _NCH, _ICH)
    quantized = _make_sc_gather()(embedding, idx_w)
    return (
        quantized,
        loss.reshape(()),
        perp.reshape(()),
    )
